# baseline (speedup 1.0000x reference)
import functools

import jax
import jax.numpy as jnp
from jax import lax
from jax.experimental import pallas as pl
from jax.experimental.pallas import tpu as pltpu
from jax.experimental.pallas import tpu_sc as plsc

_NUM_CORES = 2
_NUM_SUBCORES = 16
_NUM_WORKERS = _NUM_CORES * _NUM_SUBCORES
_LANES = 16


@functools.lru_cache(maxsize=None)
def _build(B, V, D):
    b_per_w = B // _NUM_WORKERS
    n_groups = b_per_w // _LANES
    mesh = plsc.VectorSubcoreMesh(core_axis_name="c", subcore_axis_name="s")

    @functools.partial(
        pl.kernel,
        mesh=mesh,
        out_type=jax.ShapeDtypeStruct((D, B), jnp.float32),
        compiler_params=pltpu.CompilerParams(needs_layout_passes=False),
        scratch_types=[
            pltpu.VMEM((b_per_w,), jnp.int32),
            pltpu.VMEM((b_per_w, D), jnp.float32),
            pltpu.VMEM((D, b_per_w), jnp.float32),
            pltpu.SemaphoreType.DMA,
        ],
    )
    def k(table_hbm, idx_hbm, outT_hbm, idx_v, rows_v, outT_v, sem):
        wid = lax.axis_index("s") * _NUM_CORES + lax.axis_index("c")
        base = wid * b_per_w

        pltpu.sync_copy(idx_hbm.at[pl.ds(base, b_per_w)], idx_v)

        rows_2d = rows_v

        def gather_group(g, _):
            v = idx_v[pl.ds(g * _LANES, _LANES)]
            for j in range(_LANES):
                pltpu.async_copy(
                    table_hbm.at[v[j]], rows_2d.at[g * _LANES + j], sem
                )
            return ()

        lax.fori_loop(0, n_groups, gather_group, (), unroll=2)

        pltpu.make_async_copy(
            table_hbm.at[pl.ds(0, b_per_w)], rows_2d, sem
        ).wait()

        dim_iota = lax.iota(jnp.int32, _LANES)

        def transpose_group(g, _):
            for j in range(_LANES):
                b = g * _LANES + j
                col = jnp.full((_LANES,), b, jnp.int32)
                row = jnp.full((_LANES,), b, jnp.int32)
                for kk in range(D // _LANES):
                    chunk = plsc.load_gather(
                        rows_v, [row, dim_iota + kk * _LANES]
                    )
                    plsc.store_scatter(
                        outT_v, [dim_iota + kk * _LANES, col], chunk
                    )
            return ()

        lax.fori_loop(0, n_groups, transpose_group, (), unroll=1)

        pltpu.sync_copy(outT_v, outT_hbm.at[:, pl.ds(base, b_per_w)])

    return k


def kernel(genre_id, embedding_table):
    if genre_id.ndim == 2 and genre_id.shape[1] == 1:
        genre_id = genre_id.squeeze(1)
    B = genre_id.shape[0]
    V, D = embedding_table.shape
    idx = genre_id.astype(jnp.int32)
    outT = _build(B, V, D)(embedding_table, idx)
    return outT.T
